# Initial kernel scaffold; baseline (speedup 1.0000x reference)
import functools

import jax
import jax.numpy as jnp
from jax import lax
from jax.experimental import pallas as pl
from jax.experimental.pallas import tpu as pltpu
from jax.experimental.pallas import tpu_sc as plsc

N = 10000
D = 512
B = 3500
K1 = 32
K2 = 64
NB = 7
BOT = 256
BP = 3584
NW = 32
CH = 112
BB1 = 56
G1 = BP // BB1
BB2 = 448
G2 = BP // BB2
EPS = 1e-8
PER_CLASS = B // NB


_SC_MESH = plsc.VectorSubcoreMesh(core_axis_name="c", subcore_axis_name="s")


@functools.partial(
    pl.kernel,
    mesh=_SC_MESH,
    out_type=[
        jax.ShapeDtypeStruct((BP, D), jnp.float32),
        jax.ShapeDtypeStruct((BP * K1, D), jnp.float32),
        jax.ShapeDtypeStruct((BP * K2, D), jnp.float32),
    ],
    scratch_types=[
        pltpu.VMEM((CH,), jnp.int32),
        pltpu.VMEM((CH, D), jnp.float32),
        pltpu.SemaphoreType.DMA,
    ],
)
def _sc_gather(table, idxc, idxn, idxn2, outc, outn, outn2, idx_v, rows_v, sem):
    wid = lax.axis_index("s") * 2 + lax.axis_index("c")

    def run(idx_hbm, out_hbm, chunks):
        base = wid * (chunks * CH)

        def step(i, carry):
            off = base + i * CH
            pltpu.sync_copy(idx_hbm.at[pl.ds(off, CH)], idx_v)
            pltpu.async_copy(table.at[idx_v], rows_v, sem).wait()
            pltpu.sync_copy(rows_v, out_hbm.at[pl.ds(off, CH)])
            return carry

        lax.fori_loop(0, chunks, step, 0)

    run(idxc, outc, 1)
    run(idxn, outn, K1)
    run(idxn2, outn2, K2)


def _agg_body(cen_ref, g1_ref, g2_ref, lab_ref, ws_ref, wn_ref, wn2_ref,
              w1_ref, b1_ref, w2_ref, b2_ref, raw_ref, csum_ref):
    i = pl.program_id(0)
    c = ws_ref[...] * cen_ref[...]
    g1 = wn_ref[...][:, None, :] * g1_ref[...]
    g2 = wn2_ref[...][:, None, :] * g2_ref[...]
    na = jnp.maximum(jnp.sqrt(jnp.sum(c * c, axis=-1)), EPS)
    n1 = jnp.maximum(jnp.sqrt(jnp.sum(g1 * g1, axis=-1)), EPS)
    n2 = jnp.maximum(jnp.sqrt(jnp.sum(g2 * g2, axis=-1)), EPS)
    d1 = jnp.sum(c[:, None, :] * g1, axis=-1)
    d2 = jnp.sum(c[:, None, :] * g2, axis=-1)
    s1 = d1 / (na[:, None] * n1)
    s2 = d2 / (na[:, None] * n2)
    m = jnp.maximum(jnp.max(s1, axis=-1), jnp.max(s2, axis=-1))
    e1 = jnp.exp(s1 - m[:, None])
    e2 = jnp.exp(s2 - m[:, None])
    z = jnp.sum(e1, axis=-1) + jnp.sum(e2, axis=-1)
    p1 = e1 / z[:, None]
    p2 = e2 / z[:, None]
    wsum = (jnp.sum(p1[:, :, None] * g1, axis=1)
            + jnp.sum(p2[:, :, None] * g2, axis=1))
    x = wsum + c
    h = jnp.maximum(
        jnp.dot(x, w1_ref[...], preferred_element_type=jnp.float32)
        + b1_ref[...], 0.0)
    pr = jnp.dot(h, w2_ref[...], preferred_element_type=jnp.float32) + b2_ref[...]
    raw = pr + c
    raw_ref[...] = raw
    lab = lab_ref[0]
    cls = lax.broadcasted_iota(jnp.int32, (8, BB1), 0)
    pmat = (cls == lab).astype(jnp.float32)
    part = jnp.dot(pmat, raw, preferred_element_type=jnp.float32)

    @pl.when(i == 0)
    def _init():
        csum_ref[...] = jnp.zeros_like(csum_ref)

    csum_ref[...] += part


def _agg_call(cen, g1, g2, lab3, w_self, w_nbr, w_nbr2, W1, b1, W2, b2):
    full2 = lambda shape: pl.BlockSpec(shape, lambda i: (0, 0))
    return pl.pallas_call(
        _agg_body,
        grid=(G1,),
        in_specs=[
            pl.BlockSpec((BB1, D), lambda i: (i, 0)),
            pl.BlockSpec((BB1, K1, D), lambda i: (i, 0, 0)),
            pl.BlockSpec((BB1, K2, D), lambda i: (i, 0, 0)),
            pl.BlockSpec((1, 1, BB1), lambda i: (i, 0, 0)),
            full2((1, D)), full2((1, D)), full2((1, D)),
            full2((D, BOT)), full2((1, BOT)), full2((BOT, D)), full2((1, D)),
        ],
        out_specs=[
            pl.BlockSpec((BB1, D), lambda i: (i, 0)),
            pl.BlockSpec((8, D), lambda i: (0, 0)),
        ],
        out_shape=[
            jax.ShapeDtypeStruct((BP, D), jnp.float32),
            jax.ShapeDtypeStruct((8, D), jnp.float32),
        ],
    )(cen, g1, g2, lab3, w_self, w_nbr, w_nbr2, W1, b1, W2, b2)


def _cos_body(raw_ref, csum_ref, o_ref):
    ave = csum_ref[...] * (1.0 / PER_CLASS)
    r = raw_ref[...]
    dots = lax.dot_general(r, ave, (((1,), (1,)), ((), ())),
                           preferred_element_type=jnp.float32)
    na = jnp.maximum(jnp.sqrt(jnp.sum(r * r, axis=-1)), EPS)
    nb = jnp.maximum(jnp.sqrt(jnp.sum(ave * ave, axis=-1)), EPS)
    sim = dots / (na[:, None] * nb[None, :])
    col = lax.broadcasted_iota(jnp.int32, (BB2, 8), 1)
    sim = jnp.where(col < NB, sim, -1e30)
    m = jnp.max(sim, axis=-1)
    e = jnp.exp(sim - m[:, None])
    o_ref[...] = e / jnp.sum(e, axis=-1)[:, None]


def _cos_call(raw, csum):
    return pl.pallas_call(
        _cos_body,
        grid=(G2,),
        in_specs=[
            pl.BlockSpec((BB2, D), lambda i: (i, 0)),
            pl.BlockSpec((8, D), lambda i: (0, 0)),
        ],
        out_specs=pl.BlockSpec((BB2, 8), lambda i: (i, 0)),
        out_shape=jax.ShapeDtypeStruct((BP, 8), jnp.float32),
    )(raw, csum)


def kernel(embeds, idx, neighbors, neighbors_2hop, labels, w_self, w_nbr,
           w_nbr2, W1, b1, W2, b2):
    pad = BP - B
    idxp = jnp.concatenate([idx, jnp.zeros((pad,), jnp.int32)])
    nbrp = jnp.concatenate(
        [neighbors, jnp.zeros((pad, K1), jnp.int32)]).reshape(BP * K1)
    nbr2p = jnp.concatenate(
        [neighbors_2hop, jnp.zeros((pad, K2), jnp.int32)]).reshape(BP * K2)
    lab3 = jnp.concatenate(
        [labels, jnp.full((pad,), NB, jnp.int32)]).reshape(G1, 1, BB1)
    cen, g1, g2 = _sc_gather(embeds, idxp, nbrp, nbr2p)
    g1 = g1.reshape(BP, K1, D)
    g2 = g2.reshape(BP, K2, D)
    raw, csum = _agg_call(cen, g1, g2, lab3, w_self, w_nbr, w_nbr2,
                          W1, b1.reshape(1, BOT), W2, b2.reshape(1, D))
    out = _cos_call(raw, csum)
    return out[:B, :NB]


# speedup vs baseline: 1.2026x; 1.2026x over previous
import functools

import jax
import jax.numpy as jnp
from jax import lax
from jax.experimental import pallas as pl
from jax.experimental.pallas import tpu as pltpu
from jax.experimental.pallas import tpu_sc as plsc

N = 10000
D = 512
B = 3500
K1 = 32
K2 = 64
NB = 7
BOT = 256
BP = 3584
NW = 32
CH = 112
BB1 = 56
G1 = BP // BB1
BB2 = 448
G2 = BP // BB2
EPS = 1e-8
PER_CLASS = B // NB


@functools.cache
def _sc_gather_build():
    mesh = plsc.VectorSubcoreMesh(core_axis_name="c", subcore_axis_name="s")

    @functools.partial(
        pl.kernel,
        mesh=mesh,
        out_type=[
            jax.ShapeDtypeStruct((BP, D), jnp.float32),
            jax.ShapeDtypeStruct((BP * K1, D), jnp.float32),
            jax.ShapeDtypeStruct((BP * K2, D), jnp.float32),
        ],
        scratch_types=[
            pltpu.VMEM((CH,), jnp.int32),
            pltpu.VMEM((CH, D), jnp.float32),
            pltpu.SemaphoreType.DMA,
        ],
    )
    def _sc_gather(table, idxc, idxn, idxn2, outc, outn, outn2,
                   idx_v, rows_v, sem):
        wid = lax.axis_index("s") * 2 + lax.axis_index("c")

        def run(idx_hbm, out_hbm, chunks):
            base = wid * (chunks * CH)

            def step(i, carry):
                off = base + i * CH
                pltpu.sync_copy(idx_hbm.at[pl.ds(off, CH)], idx_v)
                pltpu.async_copy(table.at[idx_v], rows_v, sem).wait()
                pltpu.sync_copy(rows_v, out_hbm.at[pl.ds(off, CH)])
                return carry

            lax.fori_loop(0, chunks, step, 0)

        run(idxc, outc, 1)
        run(idxn, outn, K1)
        run(idxn2, outn2, K2)

    return _sc_gather


def _agg_body(cen_ref, g1_ref, g2_ref, lab_ref, ws_ref, wn_ref, wn2_ref,
              w1_ref, b1_ref, w2_ref, b2_ref, raw_ref, csum_ref):
    i = pl.program_id(0)
    c = ws_ref[...] * cen_ref[...]
    g1 = wn_ref[...][:, None, :] * g1_ref[...]
    g2 = wn2_ref[...][:, None, :] * g2_ref[...]
    na = jnp.maximum(jnp.sqrt(jnp.sum(c * c, axis=-1)), EPS)
    n1 = jnp.maximum(jnp.sqrt(jnp.sum(g1 * g1, axis=-1)), EPS)
    n2 = jnp.maximum(jnp.sqrt(jnp.sum(g2 * g2, axis=-1)), EPS)
    d1 = jnp.sum(c[:, None, :] * g1, axis=-1)
    d2 = jnp.sum(c[:, None, :] * g2, axis=-1)
    s1 = d1 / (na[:, None] * n1)
    s2 = d2 / (na[:, None] * n2)
    m = jnp.maximum(jnp.max(s1, axis=-1), jnp.max(s2, axis=-1))
    e1 = jnp.exp(s1 - m[:, None])
    e2 = jnp.exp(s2 - m[:, None])
    z = jnp.sum(e1, axis=-1) + jnp.sum(e2, axis=-1)
    p1 = e1 / z[:, None]
    p2 = e2 / z[:, None]
    wsum = (jnp.sum(p1[:, :, None] * g1, axis=1)
            + jnp.sum(p2[:, :, None] * g2, axis=1))
    x = wsum + c
    h = jnp.maximum(
        jnp.dot(x, w1_ref[...], preferred_element_type=jnp.float32)
        + b1_ref[...], 0.0)
    pr = jnp.dot(h, w2_ref[...], preferred_element_type=jnp.float32) + b2_ref[...]
    raw = pr + c
    raw_ref[...] = raw
    lab = lab_ref[0]
    cls = lax.broadcasted_iota(jnp.int32, (8, BB1), 0)
    pmat = (cls == lab).astype(jnp.float32)
    part = jnp.dot(pmat, raw, preferred_element_type=jnp.float32)

    @pl.when(i == 0)
    def _init():
        csum_ref[...] = jnp.zeros_like(csum_ref)

    csum_ref[...] += part


def _agg_call(cen, g1, g2, lab3, w_self, w_nbr, w_nbr2, W1, b1, W2, b2):
    full2 = lambda shape: pl.BlockSpec(shape, lambda i: (0, 0))
    return pl.pallas_call(
        _agg_body,
        grid=(G1,),
        in_specs=[
            pl.BlockSpec((BB1, D), lambda i: (i, 0)),
            pl.BlockSpec((BB1, K1, D), lambda i: (i, 0, 0)),
            pl.BlockSpec((BB1, K2, D), lambda i: (i, 0, 0)),
            pl.BlockSpec((1, 1, BB1), lambda i: (i, 0, 0)),
            full2((1, D)), full2((1, D)), full2((1, D)),
            full2((D, BOT)), full2((1, BOT)), full2((BOT, D)), full2((1, D)),
        ],
        out_specs=[
            pl.BlockSpec((BB1, D), lambda i: (i, 0)),
            pl.BlockSpec((8, D), lambda i: (0, 0)),
        ],
        out_shape=[
            jax.ShapeDtypeStruct((BP, D), jnp.float32),
            jax.ShapeDtypeStruct((8, D), jnp.float32),
        ],
    )(cen, g1, g2, lab3, w_self, w_nbr, w_nbr2, W1, b1, W2, b2)


def _cos_body(raw_ref, csum_ref, o_ref):
    ave = csum_ref[...] * (1.0 / PER_CLASS)
    r = raw_ref[...]
    dots = lax.dot_general(r, ave, (((1,), (1,)), ((), ())),
                           preferred_element_type=jnp.float32)
    na = jnp.maximum(jnp.sqrt(jnp.sum(r * r, axis=-1)), EPS)
    nb = jnp.maximum(jnp.sqrt(jnp.sum(ave * ave, axis=-1)), EPS)
    sim = dots / (na[:, None] * nb[None, :])
    col = lax.broadcasted_iota(jnp.int32, (BB2, 8), 1)
    sim = jnp.where(col < NB, sim, -1e30)
    m = jnp.max(sim, axis=-1)
    e = jnp.exp(sim - m[:, None])
    o_ref[...] = e / jnp.sum(e, axis=-1)[:, None]


def _cos_call(raw, csum):
    return pl.pallas_call(
        _cos_body,
        grid=(G2,),
        in_specs=[
            pl.BlockSpec((BB2, D), lambda i: (i, 0)),
            pl.BlockSpec((8, D), lambda i: (0, 0)),
        ],
        out_specs=pl.BlockSpec((BB2, 8), lambda i: (i, 0)),
        out_shape=jax.ShapeDtypeStruct((BP, 8), jnp.float32),
    )(raw, csum)


def kernel(embeds, idx, neighbors, neighbors_2hop, labels, w_self, w_nbr,
           w_nbr2, W1, b1, W2, b2):
    pad = BP - B
    idxp = jnp.concatenate([idx, jnp.zeros((pad,), jnp.int32)])
    nbrp = jnp.concatenate(
        [neighbors, jnp.zeros((pad, K1), jnp.int32)]).reshape(BP * K1)
    nbr2p = jnp.concatenate(
        [neighbors_2hop, jnp.zeros((pad, K2), jnp.int32)]).reshape(BP * K2)
    lab3 = jnp.concatenate(
        [labels, jnp.full((pad,), NB, jnp.int32)]).reshape(G1, 1, BB1)
    cen, g1, g2 = _sc_gather_build()(embeds, idxp, nbrp, nbr2p)
    g1 = g1.reshape(BP, K1, D)
    g2 = g2.reshape(BP, K2, D)
    raw, csum = _agg_call(cen, g1, g2, lab3, w_self, w_nbr, w_nbr2,
                          W1, b1.reshape(1, BOT), W2, b2.reshape(1, D))
    out = _cos_call(raw, csum)
    return out[:B, :NB]


# speedup vs baseline: 1.2691x; 1.0553x over previous
import functools

import jax
import jax.numpy as jnp
from jax import lax
from jax.experimental import pallas as pl
from jax.experimental.pallas import tpu as pltpu
from jax.experimental.pallas import tpu_sc as plsc

N = 10000
D = 512
B = 3500
K1 = 32
K2 = 64
NB = 7
BOT = 256
BP = 3584
NW = 32
CH = 112
BB1 = 56
G1 = BP // BB1
BB2 = 448
G2 = BP // BB2
EPS = 1e-8
PER_CLASS = B // NB


GROWS = BP * K2 + BP * K1 + BP
CHN = 56
NCH = (GROWS // NW) // CHN
R2C = (BP * K2) // NW // CHN
R1C = (BP * K1) // NW // CHN
OFF1 = BP * K2
OFFC = BP * K2 + BP * K1


@functools.cache
def _sc_gather_build():
    mesh = plsc.VectorSubcoreMesh(core_axis_name="c", subcore_axis_name="s")

    @functools.partial(
        pl.kernel,
        mesh=mesh,
        out_type=jax.ShapeDtypeStruct((GROWS, D), jnp.float32),
        scratch_types=[
            pltpu.VMEM((NCH, CHN), jnp.int32),
            pltpu.VMEM((2, CHN, D), jnp.float32),
            pltpu.SemaphoreType.DMA,
            pltpu.SemaphoreType.DMA,
            pltpu.SemaphoreType.DMA,
        ],
    )
    def _sc_gather(table, idx2d, out_hbm, idx_v, rows_v, sem_g, sem_o0, sem_o1):
        w = lax.axis_index("s") * 2 + lax.axis_index("c")
        pltpu.sync_copy(idx2d.at[pl.ds(w * R2C, R2C)],
                        idx_v.at[pl.ds(0, R2C)])
        pltpu.sync_copy(idx2d.at[pl.ds(NW * R2C + w * R1C, R1C)],
                        idx_v.at[pl.ds(R2C, R1C)])
        pltpu.sync_copy(idx2d.at[pl.ds(NW * (R2C + R1C) + w * 2, 2)],
                        idx_v.at[pl.ds(R2C + R1C, 2)])

        def row_off(g):
            o2 = w * (R2C * CHN) + g * CHN
            o1 = OFF1 + w * (R1C * CHN) + (g - R2C) * CHN
            oc = OFFC + w * (2 * CHN) + (g - (R2C + R1C)) * CHN
            return jnp.where(g < R2C, o2, jnp.where(g < R2C + R1C, o1, oc))

        sems = (sem_o0, sem_o1)

        def gather(g, b):
            pltpu.async_copy(table.at[idx_v.at[g]], rows_v.at[b], sem_g).wait()

        def put(g, b):
            pltpu.async_copy(rows_v.at[b], out_hbm.at[pl.ds(row_off(g), CHN)],
                             sems[b])

        def drain(b):
            pltpu.make_async_copy(rows_v.at[b], out_hbm.at[pl.ds(0, CHN)],
                                  sems[b]).wait()

        gather(0, 0)
        put(0, 0)
        gather(1, 1)
        put(1, 1)

        def step(i, carry):
            g0 = 2 * i
            drain(0)
            gather(g0, 0)
            put(g0, 0)
            drain(1)
            gather(g0 + 1, 1)
            put(g0 + 1, 1)
            return carry

        lax.fori_loop(1, NCH // 2, step, 0)
        drain(0)
        drain(1)

    return _sc_gather


def _agg_body(cen_ref, g1_ref, g2_ref, lab_ref, ws_ref, wn_ref, wn2_ref,
              w1_ref, b1_ref, w2_ref, b2_ref, raw_ref, csum_ref):
    i = pl.program_id(0)
    c = ws_ref[...] * cen_ref[...]
    g1 = wn_ref[...][:, None, :] * g1_ref[...].reshape(BB1, K1, D)
    g2 = wn2_ref[...][:, None, :] * g2_ref[...].reshape(BB1, K2, D)
    na = jnp.maximum(jnp.sqrt(jnp.sum(c * c, axis=-1)), EPS)
    n1 = jnp.maximum(jnp.sqrt(jnp.sum(g1 * g1, axis=-1)), EPS)
    n2 = jnp.maximum(jnp.sqrt(jnp.sum(g2 * g2, axis=-1)), EPS)
    d1 = jnp.sum(c[:, None, :] * g1, axis=-1)
    d2 = jnp.sum(c[:, None, :] * g2, axis=-1)
    s1 = d1 / (na[:, None] * n1)
    s2 = d2 / (na[:, None] * n2)
    m = jnp.maximum(jnp.max(s1, axis=-1), jnp.max(s2, axis=-1))
    e1 = jnp.exp(s1 - m[:, None])
    e2 = jnp.exp(s2 - m[:, None])
    z = jnp.sum(e1, axis=-1) + jnp.sum(e2, axis=-1)
    p1 = e1 / z[:, None]
    p2 = e2 / z[:, None]
    wsum = (jnp.sum(p1[:, :, None] * g1, axis=1)
            + jnp.sum(p2[:, :, None] * g2, axis=1))
    x = wsum + c
    h = jnp.maximum(
        jnp.dot(x, w1_ref[...], preferred_element_type=jnp.float32)
        + b1_ref[...], 0.0)
    pr = jnp.dot(h, w2_ref[...], preferred_element_type=jnp.float32) + b2_ref[...]
    raw = pr + c
    raw_ref[...] = raw
    lab = lab_ref[0]
    cls = lax.broadcasted_iota(jnp.int32, (8, BB1), 0)
    pmat = (cls == lab).astype(jnp.float32)
    part = jnp.dot(pmat, raw, preferred_element_type=jnp.float32)

    @pl.when(i == 0)
    def _init():
        csum_ref[...] = jnp.zeros_like(csum_ref)

    csum_ref[...] += part


def _agg_call(gtab, lab3, w_self, w_nbr, w_nbr2, W1, b1, W2, b2):
    full2 = lambda shape: pl.BlockSpec(shape, lambda i: (0, 0))
    c_blk = OFFC // BB1
    n1_blk = OFF1 // (BB1 * K1)
    return pl.pallas_call(
        _agg_body,
        grid=(G1,),
        in_specs=[
            pl.BlockSpec((BB1, D), lambda i: (c_blk + i, 0)),
            pl.BlockSpec((BB1 * K1, D), lambda i: (n1_blk + i, 0)),
            pl.BlockSpec((BB1 * K2, D), lambda i: (i, 0)),
            pl.BlockSpec((1, 1, BB1), lambda i: (i, 0, 0)),
            full2((1, D)), full2((1, D)), full2((1, D)),
            full2((D, BOT)), full2((1, BOT)), full2((BOT, D)), full2((1, D)),
        ],
        out_specs=[
            pl.BlockSpec((BB1, D), lambda i: (i, 0)),
            pl.BlockSpec((8, D), lambda i: (0, 0)),
        ],
        out_shape=[
            jax.ShapeDtypeStruct((BP, D), jnp.float32),
            jax.ShapeDtypeStruct((8, D), jnp.float32),
        ],
    )(gtab, gtab, gtab, lab3, w_self, w_nbr, w_nbr2, W1, b1, W2, b2)


def _cos_body(raw_ref, csum_ref, o_ref):
    ave = csum_ref[...] * (1.0 / PER_CLASS)
    r = raw_ref[...]
    dots = lax.dot_general(r, ave, (((1,), (1,)), ((), ())),
                           preferred_element_type=jnp.float32)
    na = jnp.maximum(jnp.sqrt(jnp.sum(r * r, axis=-1)), EPS)
    nb = jnp.maximum(jnp.sqrt(jnp.sum(ave * ave, axis=-1)), EPS)
    sim = dots / (na[:, None] * nb[None, :])
    col = lax.broadcasted_iota(jnp.int32, (BB2, 8), 1)
    sim = jnp.where(col < NB, sim, -1e30)
    m = jnp.max(sim, axis=-1)
    e = jnp.exp(sim - m[:, None])
    o_ref[...] = e / jnp.sum(e, axis=-1)[:, None]


def _cos_call(raw, csum):
    return pl.pallas_call(
        _cos_body,
        grid=(G2,),
        in_specs=[
            pl.BlockSpec((BB2, D), lambda i: (i, 0)),
            pl.BlockSpec((8, D), lambda i: (0, 0)),
        ],
        out_specs=pl.BlockSpec((BB2, 8), lambda i: (i, 0)),
        out_shape=jax.ShapeDtypeStruct((BP, 8), jnp.float32),
    )(raw, csum)


def kernel(embeds, idx, neighbors, neighbors_2hop, labels, w_self, w_nbr,
           w_nbr2, W1, b1, W2, b2):
    pad = BP - B
    idxp = jnp.concatenate([idx, jnp.zeros((pad,), jnp.int32)])
    nbrp = jnp.concatenate(
        [neighbors, jnp.zeros((pad, K1), jnp.int32)]).reshape(BP * K1)
    nbr2p = jnp.concatenate(
        [neighbors_2hop, jnp.zeros((pad, K2), jnp.int32)]).reshape(BP * K2)
    lab3 = jnp.concatenate(
        [labels, jnp.full((pad,), NB, jnp.int32)]).reshape(G1, 1, BB1)
    idx2d = jnp.concatenate([nbr2p, nbrp, idxp]).reshape(GROWS // CHN, CHN)
    gtab = _sc_gather_build()(embeds, idx2d)
    raw, csum = _agg_call(gtab, lab3, w_self, w_nbr, w_nbr2,
                          W1, b1.reshape(1, BOT), W2, b2.reshape(1, D))
    out = _cos_call(raw, csum)
    return out[:B, :NB]


# speedup vs baseline: 1.6765x; 1.3210x over previous
import functools

import jax
import jax.numpy as jnp
from jax import lax
from jax.experimental import pallas as pl
from jax.experimental.pallas import tpu as pltpu
from jax.experimental.pallas import tpu_sc as plsc

N = 10000
D = 512
B = 3500
K1 = 32
K2 = 64
NB = 7
BOT = 256
BP = 3584
NW = 32
CH = 112
BB1 = 56
G1 = BP // BB1
BB2 = 448
G2 = BP // BB2
EPS = 1e-8
PER_CLASS = B // NB


DW = D // 2

GROWS = BP * K2 + BP * K1
CHN = 56
NCH = (GROWS // NW) // CHN
R2C = (BP * K2) // NW // CHN
R1C = (BP * K1) // NW // CHN
OFF1 = BP * K2
NBUF = 4


@functools.cache
def _sc_gather_build():
    mesh = plsc.VectorSubcoreMesh(core_axis_name="c", subcore_axis_name="s")

    @functools.partial(
        pl.kernel,
        mesh=mesh,
        out_type=[
            jax.ShapeDtypeStruct((GROWS, DW), jnp.int32),
            jax.ShapeDtypeStruct((BP, D), jnp.float32),
        ],
        scratch_types=[
            pltpu.VMEM((NCH, CHN), jnp.int32),
            pltpu.VMEM((2, CHN), jnp.int32),
            pltpu.VMEM((NBUF, CHN, DW), jnp.int32),
            pltpu.VMEM((CHN, D), jnp.float32),
            [pltpu.SemaphoreType.DMA] * NBUF,
            [pltpu.SemaphoreType.DMA] * NBUF,
            pltpu.SemaphoreType.DMA,
        ],
    )
    def _sc_gather(table_i32, table_f32, idxn2d, idxc2d, out_nbr, out_cen,
                   idx_v, cidx_v, rows_v, cen_v, sem_g, sem_o, sem_c):
        w = lax.axis_index("s") * 2 + lax.axis_index("c")
        pltpu.sync_copy(idxn2d.at[pl.ds(w * R2C, R2C)],
                        idx_v.at[pl.ds(0, R2C)])
        pltpu.sync_copy(idxn2d.at[pl.ds(NW * R2C + w * R1C, R1C)],
                        idx_v.at[pl.ds(R2C, R1C)])
        pltpu.sync_copy(idxc2d.at[pl.ds(w * 2, 2)], cidx_v)

        def row_off(g):
            o2 = w * (R2C * CHN) + g * CHN
            o1 = OFF1 + w * (R1C * CHN) + (g - R2C) * CHN
            return jnp.where(g < R2C, o2, o1)

        def issue_gather(g, b):
            pltpu.async_copy(table_i32.at[idx_v.at[g]], rows_v.at[b], sem_g[b])

        def wait_gather(b):
            pltpu.make_async_copy(table_i32.at[idx_v.at[0]], rows_v.at[b],
                                  sem_g[b]).wait()

        def put(g, b):
            pltpu.async_copy(rows_v.at[b],
                             out_nbr.at[pl.ds(row_off(g), CHN)], sem_o[b])

        def drain_out(b):
            pltpu.make_async_copy(rows_v.at[b], out_nbr.at[pl.ds(0, CHN)],
                                  sem_o[b]).wait()

        for b in range(NBUF):
            issue_gather(b, b)

        def step(i, carry):
            base = NBUF * i
            for b in range(NBUF):
                g = base + b
                wait_gather(b)
                put(g, b)
                nxt = g + NBUF

                @pl.when(nxt < NCH)
                def _next():
                    drain_out(b)
                    issue_gather(nxt, b)

            return carry

        lax.fori_loop(0, NCH // NBUF, step, 0)
        for b in range(NBUF):
            drain_out(b)

        for t in range(2):
            pltpu.async_copy(table_f32.at[cidx_v.at[t]], cen_v, sem_c).wait()
            pltpu.sync_copy(cen_v, out_cen.at[pl.ds(w * 112 + t * CHN, CHN)])

    return _sc_gather


def _agg_body(cen_ref, g1_ref, g2_ref, lab_ref, ws_ref, wn_ref, wn2_ref,
              w1_ref, b1_ref, w2_ref, b2_ref, raw_ref, csum_ref):
    i = pl.program_id(0)
    c = ws_ref[...] * cen_ref[...]

    def unpack(x):
        lo = lax.bitcast_convert_type(x << 16, jnp.float32)
        hi = lax.bitcast_convert_type(x & jnp.int32(-65536), jnp.float32)
        return jnp.concatenate([lo, hi], axis=-1)

    g1 = wn_ref[...][:, None, :] * unpack(g1_ref[...]).reshape(BB1, K1, D)
    g2 = wn2_ref[...][:, None, :] * unpack(g2_ref[...]).reshape(BB1, K2, D)
    na = jnp.maximum(jnp.sqrt(jnp.sum(c * c, axis=-1)), EPS)
    n1 = jnp.maximum(jnp.sqrt(jnp.sum(g1 * g1, axis=-1)), EPS)
    n2 = jnp.maximum(jnp.sqrt(jnp.sum(g2 * g2, axis=-1)), EPS)
    d1 = jnp.sum(c[:, None, :] * g1, axis=-1)
    d2 = jnp.sum(c[:, None, :] * g2, axis=-1)
    s1 = d1 / (na[:, None] * n1)
    s2 = d2 / (na[:, None] * n2)
    m = jnp.maximum(jnp.max(s1, axis=-1), jnp.max(s2, axis=-1))
    e1 = jnp.exp(s1 - m[:, None])
    e2 = jnp.exp(s2 - m[:, None])
    z = jnp.sum(e1, axis=-1) + jnp.sum(e2, axis=-1)
    p1 = e1 / z[:, None]
    p2 = e2 / z[:, None]
    wsum = (jnp.sum(p1[:, :, None] * g1, axis=1)
            + jnp.sum(p2[:, :, None] * g2, axis=1))
    x = wsum + c
    h = jnp.maximum(
        jnp.dot(x, w1_ref[...], preferred_element_type=jnp.float32)
        + b1_ref[...], 0.0)
    pr = jnp.dot(h, w2_ref[...], preferred_element_type=jnp.float32) + b2_ref[...]
    raw = pr + c
    raw_ref[...] = raw
    lab = lab_ref[0]
    cls = lax.broadcasted_iota(jnp.int32, (8, BB1), 0)
    pmat = (cls == lab).astype(jnp.float32)
    part = jnp.dot(pmat, raw, preferred_element_type=jnp.float32)

    @pl.when(i == 0)
    def _init():
        csum_ref[...] = jnp.zeros_like(csum_ref)

    csum_ref[...] += part


def _agg_call(gnbr, cen, lab3, w_self, w_nbr, w_nbr2, W1, b1, W2, b2):
    full2 = lambda shape: pl.BlockSpec(shape, lambda i: (0, 0))
    n1_blk = OFF1 // (BB1 * K1)
    return pl.pallas_call(
        _agg_body,
        grid=(G1,),
        in_specs=[
            pl.BlockSpec((BB1, D), lambda i: (i, 0)),
            pl.BlockSpec((BB1 * K1, DW), lambda i: (n1_blk + i, 0)),
            pl.BlockSpec((BB1 * K2, DW), lambda i: (i, 0)),
            pl.BlockSpec((1, 1, BB1), lambda i: (i, 0, 0)),
            full2((1, D)), full2((1, D)), full2((1, D)),
            full2((D, BOT)), full2((1, BOT)), full2((BOT, D)), full2((1, D)),
        ],
        out_specs=[
            pl.BlockSpec((BB1, D), lambda i: (i, 0)),
            pl.BlockSpec((8, D), lambda i: (0, 0)),
        ],
        out_shape=[
            jax.ShapeDtypeStruct((BP, D), jnp.float32),
            jax.ShapeDtypeStruct((8, D), jnp.float32),
        ],
    )(cen, gnbr, gnbr, lab3, w_self, w_nbr, w_nbr2, W1, b1, W2, b2)


def _cos_body(raw_ref, csum_ref, o_ref):
    ave = csum_ref[...] * (1.0 / PER_CLASS)
    r = raw_ref[...]
    dots = lax.dot_general(r, ave, (((1,), (1,)), ((), ())),
                           preferred_element_type=jnp.float32)
    na = jnp.maximum(jnp.sqrt(jnp.sum(r * r, axis=-1)), EPS)
    nb = jnp.maximum(jnp.sqrt(jnp.sum(ave * ave, axis=-1)), EPS)
    sim = dots / (na[:, None] * nb[None, :])
    col = lax.broadcasted_iota(jnp.int32, (BB2, 8), 1)
    sim = jnp.where(col < NB, sim, -1e30)
    m = jnp.max(sim, axis=-1)
    e = jnp.exp(sim - m[:, None])
    o_ref[...] = e / jnp.sum(e, axis=-1)[:, None]


def _cos_call(raw, csum):
    return pl.pallas_call(
        _cos_body,
        grid=(G2,),
        in_specs=[
            pl.BlockSpec((BB2, D), lambda i: (i, 0)),
            pl.BlockSpec((8, D), lambda i: (0, 0)),
        ],
        out_specs=pl.BlockSpec((BB2, 8), lambda i: (i, 0)),
        out_shape=jax.ShapeDtypeStruct((BP, 8), jnp.float32),
    )(raw, csum)


def kernel(embeds, idx, neighbors, neighbors_2hop, labels, w_self, w_nbr,
           w_nbr2, W1, b1, W2, b2):
    pad = BP - B
    idxp = jnp.concatenate([idx, jnp.zeros((pad,), jnp.int32)])
    nbrp = jnp.concatenate(
        [neighbors, jnp.zeros((pad, K1), jnp.int32)]).reshape(BP * K1)
    nbr2p = jnp.concatenate(
        [neighbors_2hop, jnp.zeros((pad, K2), jnp.int32)]).reshape(BP * K2)
    lab3 = jnp.concatenate(
        [labels, jnp.full((pad,), NB, jnp.int32)]).reshape(G1, 1, BB1)
    idxn2d = jnp.concatenate([nbr2p, nbrp]).reshape(GROWS // CHN, CHN)
    idxc2d = idxp.reshape(BP // CHN, CHN)
    emb_bf = embeds.astype(jnp.bfloat16)
    emb_i32 = lax.bitcast_convert_type(
        jnp.stack([emb_bf[:, :DW], emb_bf[:, DW:]], axis=-1), jnp.int32)
    gnbr, cen = _sc_gather_build()(emb_i32, embeds, idxn2d, idxc2d)
    raw, csum = _agg_call(gnbr, cen, lab3, w_self, w_nbr, w_nbr2,
                          W1, b1.reshape(1, BOT), W2, b2.reshape(1, D))
    out = _cos_call(raw, csum)
    return out[:B, :NB]


# speedup vs baseline: 1.6926x; 1.0096x over previous
import functools

import jax
import jax.numpy as jnp
from jax import lax
from jax.experimental import pallas as pl
from jax.experimental.pallas import tpu as pltpu
from jax.experimental.pallas import tpu_sc as plsc

N = 10000
D = 512
B = 3500
K1 = 32
K2 = 64
NB = 7
BOT = 256
BP = 3584
NW = 32
CH = 112
BB1 = 56
G1 = BP // BB1
BB2 = 448
G2 = BP // BB2
EPS = 1e-8
PER_CLASS = B // NB


DW = D // 2

GROWS = BP * K2 + BP * K1
OFF1 = BP * K2
CHN = 112
NCH = (GROWS // NW) // CHN
CCH = 56
NBUF = 3


@functools.cache
def _sc_gather_build():
    mesh = plsc.VectorSubcoreMesh(core_axis_name="c", subcore_axis_name="s")

    @functools.partial(
        pl.kernel,
        mesh=mesh,
        out_type=[
            jax.ShapeDtypeStruct((GROWS, DW), jnp.int32),
            jax.ShapeDtypeStruct((BP, D), jnp.float32),
        ],
        scratch_types=[
            pltpu.VMEM((NCH, CHN), jnp.int32),
            pltpu.VMEM((2, CCH), jnp.int32),
            pltpu.VMEM((NBUF, CHN, DW), jnp.int32),
            pltpu.VMEM((CCH, D), jnp.float32),
            [pltpu.SemaphoreType.DMA] * NBUF,
            [pltpu.SemaphoreType.DMA] * NBUF,
            pltpu.SemaphoreType.DMA,
        ],
    )
    def _sc_gather(table_i32, table_f32, idxn2d, idxc2d, out_nbr, out_cen,
                   idx_v, cidx_v, rows_v, cen_v, sem_g, sem_o, sem_c):
        w = lax.axis_index("s") * 2 + lax.axis_index("c")
        base = w * NCH
        pltpu.sync_copy(idxn2d.at[pl.ds(base, NCH)], idx_v)
        pltpu.sync_copy(idxc2d.at[pl.ds(w * 2, 2)], cidx_v)

        def issue_gather(l, b):
            pltpu.async_copy(table_i32.at[idx_v.at[l]], rows_v.at[b], sem_g[b])

        def wait_gather(b):
            pltpu.make_async_copy(table_i32.at[idx_v.at[0]], rows_v.at[b],
                                  sem_g[b]).wait()

        def put(l, b):
            pltpu.async_copy(rows_v.at[b],
                             out_nbr.at[pl.ds((base + l) * CHN, CHN)],
                             sem_o[b])

        def drain_out(b):
            pltpu.make_async_copy(rows_v.at[b], out_nbr.at[pl.ds(0, CHN)],
                                  sem_o[b]).wait()

        for b in range(NBUF):
            issue_gather(b, b)

        def step(i, carry):
            first = NBUF * i
            for b in range(NBUF):
                l = first + b
                wait_gather(b)
                put(l, b)
                nxt = l + NBUF

                @pl.when(nxt < NCH)
                def _next():
                    drain_out(b)
                    issue_gather(nxt, b)

            return carry

        lax.fori_loop(0, NCH // NBUF, step, 0)
        for b in range(NBUF):
            drain_out(b)

        for t in range(2):
            pltpu.async_copy(table_f32.at[cidx_v.at[t]], cen_v, sem_c).wait()
            pltpu.sync_copy(cen_v, out_cen.at[pl.ds(w * 112 + t * CCH, CCH)])

    return _sc_gather


def _agg_body(cen_ref, g1_ref, g2_ref, lab_ref, ws_ref, wn_ref, wn2_ref,
              w1_ref, b1_ref, w2_ref, b2_ref, raw_ref, csum_ref):
    i = pl.program_id(0)
    c = ws_ref[...] * cen_ref[...]

    def unpack(x):
        lo = lax.bitcast_convert_type(x << 16, jnp.float32)
        hi = lax.bitcast_convert_type(x & jnp.int32(-65536), jnp.float32)
        return jnp.concatenate([lo, hi], axis=-1)

    g1 = wn_ref[...][:, None, :] * unpack(g1_ref[...]).reshape(BB1, K1, D)
    g2 = wn2_ref[...][:, None, :] * unpack(g2_ref[...]).reshape(BB1, K2, D)
    na = jnp.maximum(jnp.sqrt(jnp.sum(c * c, axis=-1)), EPS)
    n1 = jnp.maximum(jnp.sqrt(jnp.sum(g1 * g1, axis=-1)), EPS)
    n2 = jnp.maximum(jnp.sqrt(jnp.sum(g2 * g2, axis=-1)), EPS)
    d1 = jnp.sum(c[:, None, :] * g1, axis=-1)
    d2 = jnp.sum(c[:, None, :] * g2, axis=-1)
    s1 = d1 / (na[:, None] * n1)
    s2 = d2 / (na[:, None] * n2)
    m = jnp.maximum(jnp.max(s1, axis=-1), jnp.max(s2, axis=-1))
    e1 = jnp.exp(s1 - m[:, None])
    e2 = jnp.exp(s2 - m[:, None])
    z = jnp.sum(e1, axis=-1) + jnp.sum(e2, axis=-1)
    p1 = e1 / z[:, None]
    p2 = e2 / z[:, None]
    wsum = (jnp.sum(p1[:, :, None] * g1, axis=1)
            + jnp.sum(p2[:, :, None] * g2, axis=1))
    x = wsum + c
    h = jnp.maximum(
        jnp.dot(x, w1_ref[...], preferred_element_type=jnp.float32)
        + b1_ref[...], 0.0)
    pr = jnp.dot(h, w2_ref[...], preferred_element_type=jnp.float32) + b2_ref[...]
    raw = pr + c
    raw_ref[...] = raw
    lab = lab_ref[0]
    cls = lax.broadcasted_iota(jnp.int32, (8, BB1), 0)
    pmat = (cls == lab).astype(jnp.float32)
    part = jnp.dot(pmat, raw, preferred_element_type=jnp.float32)

    @pl.when(i == 0)
    def _init():
        csum_ref[...] = jnp.zeros_like(csum_ref)

    csum_ref[...] += part


def _agg_call(gnbr, cen, lab3, w_self, w_nbr, w_nbr2, W1, b1, W2, b2):
    full2 = lambda shape: pl.BlockSpec(shape, lambda i: (0, 0))
    n1_blk = OFF1 // (BB1 * K1)
    return pl.pallas_call(
        _agg_body,
        grid=(G1,),
        in_specs=[
            pl.BlockSpec((BB1, D), lambda i: (i, 0)),
            pl.BlockSpec((BB1 * K1, DW), lambda i: (n1_blk + i, 0)),
            pl.BlockSpec((BB1 * K2, DW), lambda i: (i, 0)),
            pl.BlockSpec((1, 1, BB1), lambda i: (i, 0, 0)),
            full2((1, D)), full2((1, D)), full2((1, D)),
            full2((D, BOT)), full2((1, BOT)), full2((BOT, D)), full2((1, D)),
        ],
        out_specs=[
            pl.BlockSpec((BB1, D), lambda i: (i, 0)),
            pl.BlockSpec((8, D), lambda i: (0, 0)),
        ],
        out_shape=[
            jax.ShapeDtypeStruct((BP, D), jnp.float32),
            jax.ShapeDtypeStruct((8, D), jnp.float32),
        ],
    )(cen, gnbr, gnbr, lab3, w_self, w_nbr, w_nbr2, W1, b1, W2, b2)


def _cos_body(raw_ref, csum_ref, o_ref):
    ave = csum_ref[...] * (1.0 / PER_CLASS)
    r = raw_ref[...]
    dots = lax.dot_general(r, ave, (((1,), (1,)), ((), ())),
                           preferred_element_type=jnp.float32)
    na = jnp.maximum(jnp.sqrt(jnp.sum(r * r, axis=-1)), EPS)
    nb = jnp.maximum(jnp.sqrt(jnp.sum(ave * ave, axis=-1)), EPS)
    sim = dots / (na[:, None] * nb[None, :])
    col = lax.broadcasted_iota(jnp.int32, (BB2, 8), 1)
    sim = jnp.where(col < NB, sim, -1e30)
    m = jnp.max(sim, axis=-1)
    e = jnp.exp(sim - m[:, None])
    o_ref[...] = e / jnp.sum(e, axis=-1)[:, None]


def _cos_call(raw, csum):
    return pl.pallas_call(
        _cos_body,
        grid=(G2,),
        in_specs=[
            pl.BlockSpec((BB2, D), lambda i: (i, 0)),
            pl.BlockSpec((8, D), lambda i: (0, 0)),
        ],
        out_specs=pl.BlockSpec((BB2, 8), lambda i: (i, 0)),
        out_shape=jax.ShapeDtypeStruct((BP, 8), jnp.float32),
    )(raw, csum)


def kernel(embeds, idx, neighbors, neighbors_2hop, labels, w_self, w_nbr,
           w_nbr2, W1, b1, W2, b2):
    pad = BP - B
    idxp = jnp.concatenate([idx, jnp.zeros((pad,), jnp.int32)])
    nbrp = jnp.concatenate(
        [neighbors, jnp.zeros((pad, K1), jnp.int32)]).reshape(BP * K1)
    nbr2p = jnp.concatenate(
        [neighbors_2hop, jnp.zeros((pad, K2), jnp.int32)]).reshape(BP * K2)
    lab3 = jnp.concatenate(
        [labels, jnp.full((pad,), NB, jnp.int32)]).reshape(G1, 1, BB1)
    idxn2d = jnp.concatenate([nbr2p, nbrp]).reshape(GROWS // CHN, CHN)
    idxc2d = idxp.reshape(BP // CCH, CCH)
    emb_bf = embeds.astype(jnp.bfloat16)
    emb_i32 = lax.bitcast_convert_type(
        jnp.stack([emb_bf[:, :DW], emb_bf[:, DW:]], axis=-1), jnp.int32)
    gnbr, cen = _sc_gather_build()(emb_i32, embeds, idxn2d, idxc2d)
    raw, csum = _agg_call(gnbr, cen, lab3, w_self, w_nbr, w_nbr2,
                          W1, b1.reshape(1, BOT), W2, b2.reshape(1, D))
    out = _cos_call(raw, csum)
    return out[:B, :NB]
